# baseline (speedup 1.0000x reference)
import functools

import jax
import jax.numpy as jnp
from jax import lax
from jax.experimental import pallas as pl
from jax.experimental.pallas import tpu as pltpu
from jax.experimental.pallas import tpu_sc as plsc

NCORES = 2
NSUB = 16
NW = NCORES * NSUB
CHUNK = 128


@functools.lru_cache(maxsize=None)
def _make_sc_scatter(dt, npad, nchmax, gather):
    rows_per_tile = npad // NSUB
    mesh = plsc.VectorSubcoreMesh(core_axis_name="c", subcore_axis_name="s",
                                  num_cores=NCORES)

    @functools.partial(
        pl.kernel,
        mesh=mesh,
        out_type=jax.ShapeDtypeStruct((NCORES, npad, dt), jnp.float32),
        scratch_types=[
            pltpu.VMEM_SHARED((npad, dt), jnp.float32),
            pltpu.VMEM((16,), jnp.int32),
            pltpu.VMEM((2, CHUNK), jnp.int32),
            pltpu.VMEM((CHUNK, dt), jnp.float32),
            pltpu.SemaphoreType.DMA,
        ],
    )
    def sc_scatter(table, idx, cnt16, zeros, out, acc, c_v, i_v, rows, sem):
        c = lax.axis_index("c")
        s = lax.axis_index("s")
        wid = s * NCORES + c
        base = s * rows_per_tile
        pltpu.sync_copy(zeros.at[pl.ds(base, rows_per_tile)],
                        acc.at[pl.ds(base, rows_per_tile)])
        if not gather:
            pltpu.sync_copy(table, rows)
        pltpu.sync_copy(cnt16, c_v)
        nch_act = c_v[...][0]
        m_w = lax.div(nch_act - wid + NW - 1, NW)
        plsc.subcore_barrier()

        def chunk(j, carry):
            @pl.when(j < m_w)
            def _():
                pltpu.sync_copy(idx.at[j * NW + wid], i_v)
                if gather:
                    pltpu.async_copy(table.at[i_v.at[0]], rows, sem).wait()
                pltpu.sync_copy(rows, acc.at[i_v.at[1]], add=True)
            return carry

        lax.fori_loop(0, nchmax // NW, chunk, 0)
        plsc.subcore_barrier()
        pltpu.sync_copy(acc.at[pl.ds(base, rows_per_tile)],
                        out.at[c, pl.ds(base, rows_per_tile)])

    return sc_scatter


def _dense_layer(x, ax, ar, ac, w, b):
    n, d = x.shape
    blk = 1000
    grid = n // blk

    def body(x_ref, ax_ref, ar_ref, ac_ref, w_ref, b_ref, o_ref):
        a = ax_ref[0] + ax_ref[1]
        r = ar_ref[0] + ar_ref[1]
        cnt = ac_ref[0, :, 0:1] + ac_ref[1, :, 0:1]
        agg = (a + r) / jnp.maximum(cnt, 1.0)
        h = x_ref[...] + agg
        y = jnp.dot(h, w_ref[...], preferred_element_type=jnp.float32)
        o_ref[...] = jnp.maximum(y + b_ref[...], 0.0)

    return pl.pallas_call(
        body,
        grid=(grid,),
        in_specs=[
            pl.BlockSpec((blk, d), lambda i: (i, 0)),
            pl.BlockSpec((NCORES, blk, d), lambda i: (0, i, 0)),
            pl.BlockSpec((NCORES, blk, d), lambda i: (0, i, 0)),
            pl.BlockSpec((NCORES, blk, d), lambda i: (0, i, 0)),
            pl.BlockSpec((d, d), lambda i: (0, 0)),
            pl.BlockSpec((1, d), lambda i: (0, 0)),
        ],
        out_specs=pl.BlockSpec((blk, d), lambda i: (i, 0)),
        out_shape=jax.ShapeDtypeStruct((n, d), jnp.float32),
    )(x, ax, ar, ac, w, b)


def _pool_mlp(x, q, w1, b1, w2, b2):
    n, d = x.shape

    def body(x_ref, q_ref, w1_ref, b1_ref, w2_ref, b2_ref, o_ref):
        g = jnp.mean(x_ref[...], axis=0, keepdims=True)
        comb = jnp.concatenate([q_ref[...], g], axis=1)
        h = jnp.dot(comb, w1_ref[...], preferred_element_type=jnp.float32)
        h = jnp.maximum(h + b1_ref[...], 0.0)
        y = jnp.dot(h, w2_ref[...], preferred_element_type=jnp.float32)
        o_ref[...] = y + b2_ref[...]

    out = pl.pallas_call(
        body,
        out_shape=jax.ShapeDtypeStruct((1, d), jnp.float32),
    )(x, q, w1, b1, w2, b2)
    return out.reshape(d)


def kernel(node_features, edge_index, edge_types, distances, query_idx,
           rel_emb, Wg, bg, W1, b1, W2, b2):
    n, d = node_features.shape
    e = edge_index.shape[1]
    nlayers = Wg.shape[0]

    npad = ((n + 1 + NSUB * 8 - 1) // (NSUB * 8)) * (NSUB * 8)
    per = NW * CHUNK
    e2 = 2 * e
    cap = ((e2 + per - 1) // per) * per
    nchmax = cap // CHUNK

    ei = edge_index.astype(jnp.int32)
    src, dst = ei[0], ei[1]
    dist = distances.astype(jnp.int32)
    d_src, d_dst = dist[src], dist[dst]
    mf = d_src == d_dst + 1
    mb = d_dst == d_src + 1

    active = jnp.concatenate([mf, mb])
    s_all = jnp.concatenate([dst, src])
    g_all = jnp.concatenate([src, dst])
    et = edge_types.astype(jnp.int32)
    t_all = jnp.concatenate([et, et])
    key = (((~active).astype(jnp.uint32)) << 30) | \
        ((g_all << 7) | t_all).astype(jnp.uint32)
    k_act = jnp.sum(active.astype(jnp.int32))
    key_srt, s_srt = jax.lax.sort((key, s_all), num_keys=1, is_stable=True)
    gt_srt = (key_srt & ((1 << 21) - 1)).astype(jnp.int32)
    tail = jnp.arange(e2, dtype=jnp.int32) >= k_act
    s_cmp = jnp.where(tail, n, s_srt)
    s_cmp = jnp.concatenate([s_cmp, jnp.full((cap - e2,), n, jnp.int32)])
    gt_srt = jnp.concatenate([gt_srt, jnp.zeros((cap - e2,), jnp.int32)])
    g_cmp = gt_srt >> 7
    t_cmp = gt_srt & 127

    sc2 = s_cmp.reshape(-1, 1, CHUNK)
    idx_x = jnp.concatenate([g_cmp.reshape(-1, 1, CHUNK), sc2], axis=1)
    idx_r = jnp.concatenate([t_cmp.reshape(-1, 1, CHUNK), sc2], axis=1)
    nch_act = (k_act + CHUNK - 1) // CHUNK
    cnt16 = jnp.full((16,), nch_act, jnp.int32)

    zeros_x = jnp.zeros((npad, d), jnp.float32)
    ones_blk = jnp.ones((CHUNK, d), jnp.float32)

    re_pass = _make_sc_scatter(d, npad, nchmax, True)
    cnt_pass = _make_sc_scatter(d, npad, nchmax, False)
    x_pass = re_pass

    ar = re_pass(rel_emb, idx_r, cnt16, zeros_x)
    ac = cnt_pass(ones_blk, idx_r, cnt16, zeros_x)

    x = node_features
    for l in range(nlayers):
        ax = x_pass(x, idx_x, cnt16, zeros_x)
        x = _dense_layer(x, ax, ar, ac, Wg[l], bg[l].reshape(1, d))

    q = x[query_idx][None]
    return _pool_mlp(x, q, W1, b1.reshape(1, d), W2, b2.reshape(1, d))


# speedup vs baseline: 1.0976x; 1.0976x over previous
import functools

import jax
import jax.numpy as jnp
from jax import lax
from jax.experimental import pallas as pl
from jax.experimental.pallas import tpu as pltpu
from jax.experimental.pallas import tpu_sc as plsc

NCORES = 2
NSUB = 16
NW = NCORES * NSUB
CHUNK = 128


@functools.lru_cache(maxsize=None)
def _make_sc_scatter(dt, npad, nchmax, gather):
    rows_per_tile = npad // NSUB
    mesh = plsc.VectorSubcoreMesh(core_axis_name="c", subcore_axis_name="s",
                                  num_cores=NCORES)

    @functools.partial(
        pl.kernel,
        mesh=mesh,
        out_type=jax.ShapeDtypeStruct((NCORES, npad, dt), jnp.float32),
        scratch_types=[
            pltpu.VMEM_SHARED((npad, dt), jnp.float32),
            pltpu.VMEM((16,), jnp.int32),
            pltpu.VMEM((2, CHUNK), jnp.int32),
            pltpu.VMEM((CHUNK, dt), jnp.float32),
            pltpu.SemaphoreType.DMA,
        ],
    )
    def sc_scatter(table, idx, cnt16, zeros, out, acc, c_v, i_v, rows, sem):
        c = lax.axis_index("c")
        s = lax.axis_index("s")
        wid = s * NCORES + c
        base = s * rows_per_tile
        pltpu.sync_copy(zeros.at[pl.ds(base, rows_per_tile)],
                        acc.at[pl.ds(base, rows_per_tile)])
        if not gather:
            pltpu.sync_copy(table, rows)
        pltpu.sync_copy(cnt16, c_v)
        nch_act = c_v[...][0]
        m_w = lax.div(nch_act - wid + NW - 1, NW)
        plsc.subcore_barrier()

        def chunk(j, carry):
            @pl.when(j < m_w)
            def _():
                pltpu.sync_copy(idx.at[j * NW + wid], i_v)
                if gather:
                    pltpu.async_copy(table.at[i_v.at[0]], rows, sem).wait()
                pltpu.sync_copy(rows, acc.at[i_v.at[1]], add=True)
            return carry

        lax.fori_loop(0, nchmax // NW, chunk, 0)
        plsc.subcore_barrier()
        pltpu.sync_copy(acc.at[pl.ds(base, rows_per_tile)],
                        out.at[c, pl.ds(base, rows_per_tile)])

    return sc_scatter


def _dense_layer(x, ax, ar, ac, w, b):
    n, d = x.shape
    blk = 1000
    grid = n // blk

    def body(x_ref, ax_ref, ar_ref, ac_ref, w_ref, b_ref, o_ref):
        a = ax_ref[0] + ax_ref[1]
        r = ar_ref[0] + ar_ref[1]
        cnt = ac_ref[0, :, 0:1] + ac_ref[1, :, 0:1]
        agg = (a + r) / jnp.maximum(cnt, 1.0)
        h = x_ref[...] + agg
        y = jnp.dot(h, w_ref[...], preferred_element_type=jnp.float32)
        o_ref[...] = jnp.maximum(y + b_ref[...], 0.0)

    return pl.pallas_call(
        body,
        grid=(grid,),
        in_specs=[
            pl.BlockSpec((blk, d), lambda i: (i, 0)),
            pl.BlockSpec((NCORES, blk, d), lambda i: (0, i, 0)),
            pl.BlockSpec((NCORES, blk, d), lambda i: (0, i, 0)),
            pl.BlockSpec((NCORES, blk, d), lambda i: (0, i, 0)),
            pl.BlockSpec((d, d), lambda i: (0, 0)),
            pl.BlockSpec((1, d), lambda i: (0, 0)),
        ],
        out_specs=pl.BlockSpec((blk, d), lambda i: (i, 0)),
        out_shape=jax.ShapeDtypeStruct((n, d), jnp.float32),
    )(x, ax, ar, ac, w, b)


def _pool_mlp(x, q, w1, b1, w2, b2):
    n, d = x.shape

    def body(x_ref, q_ref, w1_ref, b1_ref, w2_ref, b2_ref, o_ref):
        g = jnp.mean(x_ref[...], axis=0, keepdims=True)
        comb = jnp.concatenate([q_ref[...], g], axis=1)
        h = jnp.dot(comb, w1_ref[...], preferred_element_type=jnp.float32)
        h = jnp.maximum(h + b1_ref[...], 0.0)
        y = jnp.dot(h, w2_ref[...], preferred_element_type=jnp.float32)
        o_ref[...] = y + b2_ref[...]

    out = pl.pallas_call(
        body,
        out_shape=jax.ShapeDtypeStruct((1, d), jnp.float32),
    )(x, q, w1, b1, w2, b2)
    return out.reshape(d)


def kernel(node_features, edge_index, edge_types, distances, query_idx,
           rel_emb, Wg, bg, W1, b1, W2, b2):
    n, d = node_features.shape
    e = edge_index.shape[1]
    nlayers = Wg.shape[0]

    npad = ((n + 1 + NSUB * 8 - 1) // (NSUB * 8)) * (NSUB * 8)
    per = NW * CHUNK
    e2 = 2 * e
    cap = ((e2 + per - 1) // per) * per
    nchmax = cap // CHUNK

    ei = edge_index.astype(jnp.int32)
    src, dst = ei[0], ei[1]
    dist = distances.astype(jnp.int32)
    d_src, d_dst = dist[src], dist[dst]
    mf = d_src == d_dst + 1
    mb = d_dst == d_src + 1

    active = jnp.concatenate([mf, mb])
    s_all = jnp.concatenate([dst, src])
    g_all = jnp.concatenate([src, dst])
    et = edge_types.astype(jnp.int32)
    t_all = jnp.concatenate([et, et])
    key = (~active).astype(jnp.int32)
    gt_all = (g_all << 7) | t_all
    k_act = jnp.sum(active.astype(jnp.int32))
    _, s_srt, gt_srt = jax.lax.sort((key, s_all, gt_all), num_keys=1,
                                    is_stable=True)
    tail = jnp.arange(e2, dtype=jnp.int32) >= k_act
    s_cmp = jnp.where(tail, n, s_srt)
    s_cmp = jnp.concatenate([s_cmp, jnp.full((cap - e2,), n, jnp.int32)])
    gt_srt = jnp.concatenate([gt_srt, jnp.zeros((cap - e2,), jnp.int32)])
    g_cmp = gt_srt >> 7
    t_cmp = gt_srt & 127

    sc2 = s_cmp.reshape(-1, 1, CHUNK)
    idx_x = jnp.concatenate([g_cmp.reshape(-1, 1, CHUNK), sc2], axis=1)
    idx_r = jnp.concatenate([t_cmp.reshape(-1, 1, CHUNK), sc2], axis=1)
    nch_act = (k_act + CHUNK - 1) // CHUNK
    cnt16 = jnp.full((16,), nch_act, jnp.int32)

    zeros_x = jnp.zeros((npad, d), jnp.float32)
    ones_blk = jnp.ones((CHUNK, d), jnp.float32)

    re_pass = _make_sc_scatter(d, npad, nchmax, True)
    cnt_pass = _make_sc_scatter(d, npad, nchmax, False)
    x_pass = re_pass

    ar = re_pass(rel_emb, idx_r, cnt16, zeros_x)
    ac = cnt_pass(ones_blk, idx_r, cnt16, zeros_x)

    x = node_features
    for l in range(nlayers):
        ax = x_pass(x, idx_x, cnt16, zeros_x)
        x = _dense_layer(x, ax, ar, ac, Wg[l], bg[l].reshape(1, d))

    q = x[query_idx][None]
    return _pool_mlp(x, q, W1, b1.reshape(1, d), W2, b2.reshape(1, d))


# speedup vs baseline: 1.1921x; 1.0861x over previous
import functools

import jax
import jax.numpy as jnp
from jax import lax
from jax.experimental import pallas as pl
from jax.experimental.pallas import tpu as pltpu
from jax.experimental.pallas import tpu_sc as plsc

NCORES = 2
NSUB = 16
NW = NCORES * NSUB
CHUNK = 128


@functools.lru_cache(maxsize=None)
def _make_sc_scatter(dt, npad, nchmax, gather):
    rows_per_tile = npad // NSUB
    mesh = plsc.VectorSubcoreMesh(core_axis_name="c", subcore_axis_name="s",
                                  num_cores=NCORES)

    @functools.partial(
        pl.kernel,
        mesh=mesh,
        out_type=jax.ShapeDtypeStruct((NCORES, npad, dt), jnp.float32),
        scratch_types=[
            pltpu.VMEM_SHARED((npad, dt), jnp.float32),
            pltpu.VMEM((16,), jnp.int32),
            pltpu.VMEM((2, CHUNK), jnp.int32),
            pltpu.VMEM((CHUNK, dt), jnp.float32),
            pltpu.SemaphoreType.DMA,
        ],
    )
    def sc_scatter(table, idx, cnt16, zeros, out, acc, c_v, i_v, rows, sem):
        c = lax.axis_index("c")
        s = lax.axis_index("s")
        wid = s * NCORES + c
        base = s * rows_per_tile
        pltpu.sync_copy(zeros.at[pl.ds(base, rows_per_tile)],
                        acc.at[pl.ds(base, rows_per_tile)])
        if not gather:
            pltpu.sync_copy(table, rows)
        pltpu.sync_copy(cnt16, c_v)
        nch_act = c_v[...][0]
        m_w = lax.div(nch_act - wid + NW - 1, NW)
        plsc.subcore_barrier()

        def chunk(j, carry):
            @pl.when(j < m_w)
            def _():
                pltpu.sync_copy(idx.at[j * NW + wid], i_v)
                if gather:
                    pltpu.async_copy(table.at[i_v.at[0]], rows, sem).wait()
                pltpu.sync_copy(rows, acc.at[i_v.at[1]], add=True)
            return carry

        lax.fori_loop(0, nchmax // NW, chunk, 0)
        plsc.subcore_barrier()
        pltpu.sync_copy(acc.at[pl.ds(base, rows_per_tile)],
                        out.at[c, pl.ds(base, rows_per_tile)])

    return sc_scatter


def _dense_layer(x, ax, ar, ac, w, b):
    n, d = x.shape
    blk = 1000
    grid = n // blk

    def body(x_ref, ax_ref, ar_ref, ac_ref, w_ref, b_ref, o_ref):
        a = ax_ref[0] + ax_ref[1]
        r = ar_ref[0] + ar_ref[1]
        cnt = ac_ref[0, :, 0:1] + ac_ref[1, :, 0:1]
        agg = (a + r) / jnp.maximum(cnt, 1.0)
        h = x_ref[...] + agg
        y = jnp.dot(h, w_ref[...], preferred_element_type=jnp.float32)
        o_ref[...] = jnp.maximum(y + b_ref[...], 0.0)

    return pl.pallas_call(
        body,
        grid=(grid,),
        in_specs=[
            pl.BlockSpec((blk, d), lambda i: (i, 0)),
            pl.BlockSpec((NCORES, blk, d), lambda i: (0, i, 0)),
            pl.BlockSpec((NCORES, blk, d), lambda i: (0, i, 0)),
            pl.BlockSpec((NCORES, blk, d), lambda i: (0, i, 0)),
            pl.BlockSpec((d, d), lambda i: (0, 0)),
            pl.BlockSpec((1, d), lambda i: (0, 0)),
        ],
        out_specs=pl.BlockSpec((blk, d), lambda i: (i, 0)),
        out_shape=jax.ShapeDtypeStruct((n, d), jnp.float32),
    )(x, ax, ar, ac, w, b)


def _pool_mlp(x, q, w1, b1, w2, b2):
    n, d = x.shape

    def body(x_ref, q_ref, w1_ref, b1_ref, w2_ref, b2_ref, o_ref):
        g = jnp.mean(x_ref[...], axis=0, keepdims=True)
        comb = jnp.concatenate([q_ref[...], g], axis=1)
        h = jnp.dot(comb, w1_ref[...], preferred_element_type=jnp.float32)
        h = jnp.maximum(h + b1_ref[...], 0.0)
        y = jnp.dot(h, w2_ref[...], preferred_element_type=jnp.float32)
        o_ref[...] = y + b2_ref[...]

    out = pl.pallas_call(
        body,
        out_shape=jax.ShapeDtypeStruct((1, d), jnp.float32),
    )(x, q, w1, b1, w2, b2)
    return out.reshape(d)


def kernel(node_features, edge_index, edge_types, distances, query_idx,
           rel_emb, Wg, bg, W1, b1, W2, b2):
    n, d = node_features.shape
    e = edge_index.shape[1]
    nlayers = Wg.shape[0]

    npad = ((n + 1 + NSUB * 8 - 1) // (NSUB * 8)) * (NSUB * 8)
    per = NW * CHUNK
    cap = ((e + per - 1) // per) * per
    nchmax = cap // CHUNK

    ei = edge_index.astype(jnp.int32)
    src, dst = ei[0], ei[1]
    dist = distances.astype(jnp.int32)
    d_src, d_dst = dist[src], dist[dst]
    mf = d_src == d_dst + 1
    mb = d_dst == d_src + 1

    et = edge_types.astype(jnp.int32)
    key = jnp.where(mf, 0, jnp.where(mb, 1, 2)).astype(jnp.int32)
    kf = jnp.sum(mf.astype(jnp.int32))
    k_act = kf + jnp.sum(mb.astype(jnp.int32))
    _, o_src, o_dst, o_t = jax.lax.sort((key, src, dst, et), num_keys=1,
                                        is_stable=True)
    i_e = jnp.arange(e, dtype=jnp.int32)
    s_cmp = jnp.where(i_e < kf, o_dst, jnp.where(i_e < k_act, o_src, n))
    g_cmp = jnp.where(i_e < kf, o_src, o_dst)
    s_cmp = jnp.concatenate([s_cmp, jnp.full((cap - e,), n, jnp.int32)])
    g_cmp = jnp.concatenate([g_cmp, jnp.zeros((cap - e,), jnp.int32)])
    t_cmp = jnp.concatenate([o_t, jnp.zeros((cap - e,), jnp.int32)])

    sc2 = s_cmp.reshape(-1, 1, CHUNK)
    idx_x = jnp.concatenate([g_cmp.reshape(-1, 1, CHUNK), sc2], axis=1)
    idx_r = jnp.concatenate([t_cmp.reshape(-1, 1, CHUNK), sc2], axis=1)
    nch_act = (k_act + CHUNK - 1) // CHUNK
    cnt16 = jnp.full((16,), nch_act, jnp.int32)

    zeros_x = jnp.zeros((npad, d), jnp.float32)
    ones_blk = jnp.ones((CHUNK, d), jnp.float32)

    re_pass = _make_sc_scatter(d, npad, nchmax, True)
    cnt_pass = _make_sc_scatter(d, npad, nchmax, False)
    x_pass = re_pass

    ar = re_pass(rel_emb, idx_r, cnt16, zeros_x)
    ac = cnt_pass(ones_blk, idx_r, cnt16, zeros_x)

    x = node_features
    for l in range(nlayers):
        ax = x_pass(x, idx_x, cnt16, zeros_x)
        x = _dense_layer(x, ax, ar, ac, Wg[l], bg[l].reshape(1, d))

    q = x[query_idx][None]
    return _pool_mlp(x, q, W1, b1.reshape(1, d), W2, b2.reshape(1, d))
